# Initial kernel scaffold; baseline (speedup 1.0000x reference)
import jax
import jax.numpy as jnp
from jax.experimental import pallas as pl


def kernel(outputs, targets):
    raise NotImplementedError("write your pallas kernel here")


# speedup vs baseline: 28.2961x; 28.2961x over previous
import functools

import jax
import jax.numpy as jnp
from jax import lax
from jax.experimental import pallas as pl
from jax.experimental.pallas import tpu as pltpu
from jax.experimental.pallas import tpu_sc as plsc

NPTS = 524288
NCLS = 20
K = 2048
NC, NS, L = 2, 16, 16
NW = NC * NS
PPW = NPTS // NW
P = 2048
NCHUNK = PPW // P
GROUPS = P // L
HW = NCLS * K
LN2 = 0.6931471805599453


def _log_ge1(s):
    bits = plsc.bitcast(s, jnp.int32)
    e = ((bits >> 23) & 0xFF) - 127
    man = plsc.bitcast((bits & 0x007FFFFF) | 0x3F800000, jnp.float32)
    z = (man - 1.0) / (man + 1.0)
    z2 = z * z
    p = jnp.float32(1.0 / 9.0)
    p = p * z2 + jnp.float32(1.0 / 7.0)
    p = p * z2 + jnp.float32(1.0 / 5.0)
    p = p * z2 + jnp.float32(1.0 / 3.0)
    p = p * z2 + jnp.float32(1.0)
    return e.astype(jnp.float32) * jnp.float32(LN2) + jnp.float32(2.0) * z * p


def _hist_body(outs_ref, tgts_ref, hist_out, ce_out, hist, buf, tbuf, cebuf):
    cid = lax.axis_index("c")
    sid = lax.axis_index("s")
    wid = sid * NC + cid
    base = wid * PPW
    iota = lax.broadcasted_iota(jnp.int32, (L,), 0)
    zero16i = jnp.zeros((L,), jnp.int32)

    def zbody(i, _):
        hist[pl.ds(i * L, L)] = zero16i
        return 0

    lax.fori_loop(0, HW // L, zbody, 0)

    def chunk_body(ci, carry):
        off = base + ci * P
        pltpu.sync_copy(outs_ref.at[pl.ds(off * NCLS, P * NCLS)], buf)
        pltpu.sync_copy(tgts_ref.at[pl.ds(off, P)], tbuf)

        def group_body(g, carry2):
            ce_a, vc_a = carry2
            rows = (g * L + iota) * NCLS
            tgt = tbuf[pl.ds(g * L, L)]
            vals = [plsc.load_gather(buf, [rows + c]) for c in range(NCLS)]
            m = vals[0]
            for c in range(1, NCLS):
                m = jnp.maximum(m, vals[c])
            es = [jnp.exp(v - m) for v in vals]
            s = es[0]
            for c in range(1, NCLS):
                s = s + es[c]
            o_t = plsc.load_gather(buf, [rows + tgt])
            validf = jnp.where(tgt != 0, jnp.float32(1.0), jnp.float32(0.0))
            nll = m + _log_ge1(s) - o_t
            ce_a = ce_a + nll * validf
            vc_a = vc_a + validf
            rinv = jnp.float32(K) / s
            kcap = jnp.full((L,), K - 1, jnp.int32)
            for c in range(NCLS):
                x = es[c] * rinv
                isfg = tgt == c
                xe = jnp.where(isfg, jnp.float32(K) - x, x)
                kb = jnp.minimum(xe.astype(jnp.int32), kcap)
                val = jnp.where(isfg, jnp.int32(65537), jnp.int32(65536))
                plsc.addupdate_scatter(hist, [kb + (c * K)], val)
            return ce_a, vc_a

        return lax.fori_loop(0, GROUPS, group_body, carry)

    zf = jnp.zeros((L,), jnp.float32)
    ce_acc, vc_acc = lax.fori_loop(0, NCHUNK, chunk_body, (zf, zf))
    cebuf[pl.ds(0, L)] = ce_acc
    cebuf[pl.ds(L, L)] = vc_acc
    pltpu.sync_copy(hist, hist_out.at[pl.ds(wid * HW, HW)])
    pltpu.sync_copy(cebuf, ce_out.at[pl.ds(wid * 2 * L, 2 * L)])


def _finish_body(hist_in, ce_in, out_hbm, hsl, accn, accfg, cebuf, outbuf, sem):
    cid = lax.axis_index("c")
    sid = lax.axis_index("s")
    wid = sid * NC + cid
    iota = lax.broadcasted_iota(jnp.int32, (L,), 0)
    zero16i = jnp.zeros((L,), jnp.int32)

    @pl.when(wid < NCLS)
    def _class_work():
        c = wid
        cps = [
            pltpu.async_copy(
                hist_in.at[pl.ds(t * HW + c * K, K)],
                hsl.at[pl.ds(t * K, K)], sem)
            for t in range(NW)
        ]
        for cp in cps:
            cp.wait()

        def merge_body(j, gacc):
            vn = zero16i
            vf = zero16i
            for t in range(NW):
                v = hsl[pl.ds(t * K + j * L, L)]
                vn = vn + (v >> 16)
                vf = vf + (v & 0xFFFF)
            accn[pl.ds(j * L, L)] = vn
            accfg[pl.ds(j * L, L)] = vf
            return gacc + vf

        gacc = lax.fori_loop(0, K // L, merge_body, zero16i)
        g_i = jnp.sum(gacc)
        gf = g_i.astype(jnp.float32)

        def cum_body(t, carry):
            cb_c, cB_c, lacc = carry
            j = (K // L - 1) - t
            rn = jnp.flip(accn[pl.ds(j * L, L)], 0)
            rf = jnp.flip(accfg[pl.ds(j * L, L)], 0)
            cb = plsc.cumsum(rn) + cb_c
            cB = plsc.cumsum(rf) + cB_c
            b = cb.astype(jnp.float32)
            B = cB.astype(jnp.float32)
            a = b - rn.astype(jnp.float32)
            A = B - rf.astype(jnp.float32)
            jb = 1.0 - (gf - B) / (gf + b - B)
            ja = 1.0 - (gf - A) / (gf + a - A)
            krev = (j * L + (L - 1)) - iota
            ebar = (krev.astype(jnp.float32) + 0.5) * jnp.float32(1.0 / K)
            lacc = lacc + ebar * (jb - ja)
            return jnp.max(cb), jnp.max(cB), lacc

        _, _, lacc = lax.fori_loop(
            0, K // L, cum_body,
            (jnp.int32(0), jnp.int32(0), jnp.zeros((L,), jnp.float32)))
        loss = jnp.sum(lacc)
        present = g_i > 0
        loss_m = jnp.where(present, loss, jnp.float32(0.0))
        presf = jnp.where(present, jnp.float32(1.0), jnp.float32(0.0))
        w = jnp.where(iota == 0, loss_m,
                      jnp.where(iota == 1, presf, jnp.float32(0.0)))
        outbuf[pl.ds(0, L)] = w
        pltpu.sync_copy(outbuf, out_hbm.at[pl.ds(c * L, L)])

    @pl.when(wid == NCLS)
    def _ce_work():
        pltpu.sync_copy(ce_in, cebuf)
        zf = jnp.zeros((L,), jnp.float32)

        def sum_body(t, carry):
            ca, va = carry
            return (ca + cebuf[pl.ds(t * 2 * L, L)],
                    va + cebuf[pl.ds(t * 2 * L + L, L)])

        ca, va = lax.fori_loop(0, NW, sum_body, (zf, zf))
        outbuf[pl.ds(0, L)] = ca
        pltpu.sync_copy(outbuf, out_hbm.at[pl.ds(NCLS * L, L)])
        outbuf[pl.ds(0, L)] = va
        pltpu.sync_copy(outbuf, out_hbm.at[pl.ds((NCLS + 1) * L, L)])


_mesh = plsc.VectorSubcoreMesh(
    core_axis_name="c", subcore_axis_name="s", num_cores=NC, num_subcores=NS)

_params = pltpu.CompilerParams(needs_layout_passes=False)

_hist_kernel = pl.kernel(
    _hist_body,
    out_type=(
        jax.ShapeDtypeStruct((NW * HW,), jnp.int32),
        jax.ShapeDtypeStruct((NW * 2 * L,), jnp.float32),
    ),
    mesh=_mesh,
    compiler_params=_params,
    scratch_types=(
        pltpu.VMEM((HW,), jnp.int32),
        pltpu.VMEM((P * NCLS,), jnp.float32),
        pltpu.VMEM((P,), jnp.int32),
        pltpu.VMEM((2 * L,), jnp.float32),
    ),
)

_finish_kernel = pl.kernel(
    _finish_body,
    out_type=jax.ShapeDtypeStruct(((NCLS + 2) * L,), jnp.float32),
    mesh=_mesh,
    compiler_params=_params,
    scratch_types=(
        pltpu.VMEM((NW * K,), jnp.int32),
        pltpu.VMEM((K,), jnp.int32),
        pltpu.VMEM((K,), jnp.int32),
        pltpu.VMEM((NW * 2 * L,), jnp.float32),
        pltpu.VMEM((L,), jnp.float32),
        pltpu.SemaphoreType.DMA,
    ),
)


def kernel(outputs, targets):
    hist, cestat = _hist_kernel(
        outputs.reshape(-1), targets.astype(jnp.int32))
    out = _finish_kernel(hist, cestat).reshape(NCLS + 2, L)
    loss = out[:NCLS, 0]
    cnt = out[:NCLS, 1].sum()
    lovasz = jnp.where(cnt > 0, loss.sum() / jnp.maximum(cnt, 1.0),
                       jnp.float32(0.0))
    ce = out[NCLS].sum() / out[NCLS + 1].sum()
    return (ce, lovasz)
